# baseline (speedup 1.0000x reference)
import functools

import jax
import jax.numpy as jnp
from jax import lax
from jax.experimental import pallas as pl
from jax.experimental.pallas import tpu as pltpu
from jax.experimental.pallas import tpu_sc as plsc

_f32 = jnp.float32
_i32 = jnp.int32

_NC, _NS = 2, 16
_D = 128
_B = 64
_NPAD = 10240
_RPT = _NPAD // _NS
_NQ = _NPAD // 16
_QPT = _NQ // _NS
_EC = 128
_K = 80
_CB = 8
_NCH = _K // _CB
_EPT = _K * _EC
_EPAD = _NC * _NS * _EPT
_NBLK = 8
_RB = _NPAD // _NBLK

_mesh = plsc.VectorSubcoreMesh(
    core_axis_name="c", subcore_axis_name="s", num_cores=_NC, num_subcores=_NS
)
_sc_params = pltpu.CompilerParams(needs_layout_passes=False)

def _bcast(v, lane):
    idx = jnp.full((16,), lane, _i32)
    return v.at[idx].get(mode="promise_in_bounds")


@functools.partial(
    pl.kernel,
    out_type=jax.ShapeDtypeStruct((_NC, _NQ, _D), _f32),
    mesh=_mesh,
    compiler_params=_sc_params,
    scratch_types=[
        pltpu.VMEM_SHARED((_NQ, _D), _f32),
        pltpu.VMEM((_K, _EC), _i32),
        pltpu.VMEM((_K, _EC), _i32),
        pltpu.VMEM((_K, _EC), _f32),
        pltpu.VMEM((_EC, _D), _f32),
        pltpu.VMEM((_QPT, _D), _f32),
    ],
)
def _deg(col_hbm, colq_hbm, ew_hbm, deg_out, acc16, colbuf, colqbuf, ewbuf,
         spread, degbuf):
    c = lax.axis_index("c")
    s = lax.axis_index("s")
    fiota = lax.iota(_i32, 16).astype(_f32)
    zeros16 = jnp.zeros((16,), _f32)

    def zdeg(i, carry):
        for g in range(8):
            degbuf[i, pl.ds(g * 16, 16)] = zeros16
        return carry

    lax.fori_loop(0, _QPT, zdeg, 0)
    pltpu.sync_copy(degbuf, acc16.at[pl.ds(s * _QPT, _QPT)])

    def zspread(i, carry):
        for g in range(8):
            spread[i, pl.ds(g * 16, 16)] = zeros16
        return carry

    lax.fori_loop(0, _EC, zspread, 0)
    plsc.subcore_barrier()

    pltpu.sync_copy(col_hbm.at[c, s], colbuf)
    pltpu.sync_copy(colq_hbm.at[c, s], colqbuf)
    pltpu.sync_copy(ew_hbm.at[c, s], ewbuf)

    def dstep(j, carry):
        for g in range(8):
            colg = colbuf[j, pl.ds(g * 16, 16)]
            ewg = ewbuf[j, pl.ds(g * 16, 16)]
            lowf = jnp.bitwise_and(colg, 15).astype(_f32)
            for lane in range(16):
                m = fiota == _bcast(lowf, lane)
                spread[g * 16 + lane, pl.ds(0, 16)] = jnp.where(
                    m, _bcast(ewg, lane), 0.0)
        pltpu.sync_copy(spread, acc16.at[colqbuf.at[j]], add=True)
        return carry

    lax.fori_loop(0, _K, dstep, 0)
    plsc.subcore_barrier()
    pltpu.sync_copy(acc16.at[pl.ds(s * _QPT, _QPT)], degbuf)
    pltpu.sync_copy(degbuf, deg_out.at[c, pl.ds(s * _QPT, _QPT)])


def _dinv_body(d_ref, dinv_ref, selfc_ref):
    deg = d_ref[0] + d_ref[1] + 2.0
    y = jnp.where(deg > 0, lax.rsqrt(jnp.where(deg > 0, deg, 1.0)), 0.0)
    dinv_ref[...] = y
    selfc_ref[...] = 2.0 * y * y


_dinv = pl.pallas_call(
    _dinv_body,
    out_shape=(
        jax.ShapeDtypeStruct((_NPAD // _D, _D), _f32),
        jax.ShapeDtypeStruct((_NPAD // _D, _D), _f32),
    ),
)


@functools.partial(
    pl.kernel,
    out_type=jax.ShapeDtypeStruct((_NC, _NS, _K, _EC), _f32),
    mesh=_mesh,
    compiler_params=_sc_params,
    scratch_types=[
        pltpu.VMEM((_K, _EC), _i32),
        pltpu.VMEM((_K, _EC), _i32),
        pltpu.VMEM((_K, _EC), _f32),
        pltpu.VMEM((_NPAD,), _f32),
        pltpu.VMEM((_K, _EC), _f32),
    ],
)
def _norm(row_hbm, col_hbm, ew_hbm, dinv_hbm, norm_out,
          rowbuf, colbuf, ewbuf, dinv_full, normbuf):
    c = lax.axis_index("c")
    s = lax.axis_index("s")
    pltpu.sync_copy(dinv_hbm, dinv_full)
    pltpu.sync_copy(row_hbm.at[c, s], rowbuf)
    pltpu.sync_copy(col_hbm.at[c, s], colbuf)
    pltpu.sync_copy(ew_hbm.at[c, s], ewbuf)

    def nstep(j, carry):
        for g in range(8):
            r = rowbuf[j, pl.ds(g * 16, 16)]
            cc = colbuf[j, pl.ds(g * 16, 16)]
            ew = ewbuf[j, pl.ds(g * 16, 16)]
            dr = plsc.load_gather(dinv_full, [r])
            dc = plsc.load_gather(dinv_full, [cc])
            normbuf[j, pl.ds(g * 16, 16)] = dr * ew * dc
        return carry

    lax.fori_loop(0, _K, nstep, 0)
    pltpu.sync_copy(normbuf, norm_out.at[c, s])


@functools.partial(
    pl.kernel,
    out_type=jax.ShapeDtypeStruct((_NC, _NPAD, _D), _f32),
    mesh=_mesh,
    compiler_params=_sc_params,
    scratch_types=[
        pltpu.VMEM_SHARED((_NPAD, _D), _f32),
        pltpu.VMEM((2 * _CB, _EC), _i32),
        pltpu.VMEM((_CB, _EC), _f32),
        pltpu.VMEM((_EC, _D), _f32),
        pltpu.VMEM((_EC, _D), _f32),
        pltpu.VMEM((_RPT,), _f32),
        pltpu.VMEM((_D,), _f32),
        pltpu.SemaphoreType.DMA,
        pltpu.SemaphoreType.DMA,
        pltpu.SemaphoreType.DMA,
        pltpu.SemaphoreType.DMA,
    ],
)
def _scat(y_hbm, idx_hbm, norm_hbm, selfc_hbm, bias_hbm,
          p_hbm, acc, ibuf, nbuf, msg0, msg1, selfcbuf,
          biasbuf, gsem0, gsem1, ssem0, ssem1):
    c = lax.axis_index("c")
    s = lax.axis_index("s")
    pltpu.sync_copy(bias_hbm, biasbuf)
    flag = jnp.where(c == 0, 1.0, 0.0).astype(_f32)
    bias_vs = [biasbuf[pl.ds(g * 16, 16)] * flag for g in range(8)]
    pltpu.sync_copy(selfc_hbm.at[pl.ds(s * _RPT, _RPT)], selfcbuf)

    def ichunk(chunk, carry):
        base = s * _RPT + chunk * _EC
        pltpu.sync_copy(y_hbm.at[pl.ds(base, _EC)], msg0)

        def irow(gg, carry2):
            sv = selfcbuf[pl.ds(chunk * _EC + gg * 16, 16)] * flag
            for lane in range(16):
                sc = _bcast(sv, lane)
                e = gg * 16 + lane
                for g in range(8):
                    msg0[e, pl.ds(g * 16, 16)] = (
                        msg0[e, pl.ds(g * 16, 16)] * sc + bias_vs[g])
            return carry2

        lax.fori_loop(0, _EC // 16, irow, 0)
        pltpu.sync_copy(msg0, acc.at[pl.ds(base, _EC)])
        return carry

    lax.fori_loop(0, _RPT // _EC, ichunk, 0)
    plsc.subcore_barrier()

    bufs = (msg0, msg1)
    gsems = (gsem0, gsem1)
    ssems = (ssem0, ssem1)

    def _scale(mb, t):
        def srow(gg, carry2):
            nv = nbuf[t, pl.ds(gg * 16, 16)]
            for lane in range(16):
                nb = _bcast(nv, lane)
                e = gg * 16 + lane
                for g in range(8):
                    mb[e, pl.ds(g * 16, 16)] = mb[e, pl.ds(g * 16, 16)] * nb
            return carry2

        lax.fori_loop(0, _EC // 16, srow, 0)

    def _slot(t, wait_scat, do_gather):
        b = t % 2
        b2 = (b + 1) % 2
        if wait_scat:
            pltpu.make_async_copy(
                bufs[b2], acc.at[ibuf.at[_CB]], ssems[b2]).wait()
        if do_gather:
            pltpu.async_copy(y_hbm.at[ibuf.at[t + 1]], bufs[b2], gsems[b2])
        pltpu.make_async_copy(y_hbm.at[ibuf.at[t]], bufs[b], gsems[b]).wait()
        _scale(bufs[b], t)
        pltpu.async_copy(bufs[b], acc.at[ibuf.at[_CB + t]], ssems[b],
                         add=True)

    def _chunk(q, first):
        if not first:
            pltpu.make_async_copy(msg1, acc.at[ibuf.at[_CB]], ssem1).wait()
        pltpu.sync_copy(idx_hbm.at[c, s, q], ibuf)
        pltpu.sync_copy(norm_hbm.at[c, s, q], nbuf)
        pltpu.async_copy(y_hbm.at[ibuf.at[0]], msg0, gsem0)

        def pair(t2, carry):
            t0 = 2 * t2

            def dyn_slot(t, b):
                b2 = (b + 1) % 2
                pltpu.make_async_copy(
                    bufs[b2], acc.at[ibuf.at[_CB]], ssems[b2]).wait()
                pltpu.async_copy(y_hbm.at[ibuf.at[t + 1]], bufs[b2],
                                 gsems[b2])
                pltpu.make_async_copy(
                    y_hbm.at[ibuf.at[t]], bufs[b], gsems[b]).wait()

                def srow(gg, carry2):
                    nv = nbuf[t, pl.ds(gg * 16, 16)]
                    mb = bufs[b]
                    for lane in range(16):
                        nb = _bcast(nv, lane)
                        e = gg * 16 + lane
                        for g in range(8):
                            mb[e, pl.ds(g * 16, 16)] = (
                                mb[e, pl.ds(g * 16, 16)] * nb)
                    return carry2

                lax.fori_loop(0, _EC // 16, srow, 0)
                pltpu.async_copy(bufs[b], acc.at[ibuf.at[_CB + t]], ssems[b],
                                 add=True)

            dyn_slot(t0, 0)
            dyn_slot(t0 + 1, 1)
            return carry

        _slot(0, False, True)
        _slot(1, True, True)
        lax.fori_loop(1, _CB // 2 - 1, pair, 0)
        _slot(_CB - 2, True, True)
        _slot(_CB - 1, True, False)

    _chunk(0, True)

    def qloop(q, carry):
        _chunk(q, False)
        return carry

    lax.fori_loop(1, _NCH, qloop, 0)
    pltpu.make_async_copy(msg1, acc.at[ibuf.at[_CB]], ssem1).wait()
    plsc.subcore_barrier()
    pltpu.sync_copy(acc.at[pl.ds(s * _RPT, _RPT)],
                    p_hbm.at[c, pl.ds(s * _RPT, _RPT)])


def _mm_body(x_ref, w_ref, o_ref):
    o_ref[...] = jnp.dot(x_ref[...], w_ref[...], preferred_element_type=_f32)


_mm = pl.pallas_call(
    _mm_body,
    grid=(_NBLK,),
    in_specs=[
        pl.BlockSpec((_RB, _D), lambda i: (i, 0)),
        pl.BlockSpec((_D, _D), lambda i: (0, 0)),
    ],
    out_specs=pl.BlockSpec((_RB, _D), lambda i: (i, 0)),
    out_shape=jax.ShapeDtypeStruct((_NPAD, _D), _f32),
)


def _mm2_body(p0_ref, p1_ref, w_ref, o_ref):
    h = p0_ref[...] + p1_ref[...]
    o_ref[...] = jnp.dot(h, w_ref[...], preferred_element_type=_f32)


_mm2 = pl.pallas_call(
    _mm2_body,
    grid=(_NBLK,),
    in_specs=[
        pl.BlockSpec((_RB, _D), lambda i: (i, 0)),
        pl.BlockSpec((_RB, _D), lambda i: (i, 0)),
        pl.BlockSpec((_D, _D), lambda i: (0, 0)),
    ],
    out_specs=pl.BlockSpec((_RB, _D), lambda i: (i, 0)),
    out_shape=jax.ShapeDtypeStruct((_NPAD, _D), _f32),
)


def _pool_body(p0_ref, p1_ref, b_ref, o_ref, cnt_ref):
    i = pl.program_id(0)

    @pl.when(i == 0)
    def _():
        o_ref[...] = jnp.zeros_like(o_ref)
        cnt_ref[...] = jnp.zeros_like(cnt_ref)

    h = p0_ref[...] + p1_ref[...]
    ids = b_ref[0]
    oh = (lax.broadcasted_iota(_i32, (_B, _RB), 0) == ids).astype(_f32)
    o_ref[...] += jnp.dot(oh, h, preferred_element_type=_f32)
    cnt_ref[...] += jnp.dot(oh, jnp.ones_like(h), preferred_element_type=_f32)

    @pl.when(i == pl.num_programs(0) - 1)
    def _():
        o_ref[...] = o_ref[...] / jnp.maximum(cnt_ref[...], 1.0)


_pool = pl.pallas_call(
    _pool_body,
    grid=(_NBLK,),
    in_specs=[
        pl.BlockSpec((_RB, _D), lambda i: (i, 0)),
        pl.BlockSpec((_RB, _D), lambda i: (i, 0)),
        pl.BlockSpec((1, 1, _RB), lambda i: (i, 0, 0)),
    ],
    out_specs=pl.BlockSpec((_B, _D), lambda i: (0, 0)),
    out_shape=jax.ShapeDtypeStruct((_B, _D), _f32),
    scratch_shapes=[pltpu.VMEM((_B, _D), _f32)],
)


def kernel(x, edge_index, edge_attr, batch, W0, b0, W1, b1, W2, b2):
    n = x.shape[0]
    e = edge_index.shape[1]
    x_pad = jnp.pad(x, ((0, _NPAD - n), (0, 0)))
    ew = jnp.reshape(edge_attr, (-1,))
    pe = _EPAD - e
    row_s = jnp.pad(edge_index[0], (0, pe)).astype(_i32).reshape(
        _NC, _NS, _K, _EC)
    col_s = jnp.pad(edge_index[1], (0, pe)).astype(_i32).reshape(
        _NC, _NS, _K, _EC)
    ew_s = jnp.pad(ew, (0, pe)).reshape(_NC, _NS, _K, _EC)
    row4 = row_s.reshape(_NC, _NS, _NCH, _CB, _EC)
    col4 = col_s.reshape(_NC, _NS, _NCH, _CB, _EC)
    idx_s = jnp.concatenate([row4, col4], axis=3)
    batch_p = jnp.pad(batch.astype(_i32), (0, _NPAD - n),
                      constant_values=_B).reshape(_NBLK, 1, _RB)

    colq_s = lax.shift_right_logical(col_s, 4)
    deg2 = _deg(col_s, colq_s, ew_s)
    deg2 = deg2[:, :, :16].reshape(_NC, _NPAD // _D, _D)
    dinv, selfc = _dinv(deg2)
    dinv = dinv.reshape(_NPAD)
    selfc = selfc.reshape(_NPAD)
    norm_s = _norm(row_s, col_s, ew_s, dinv)
    norm_c = norm_s.reshape(_NC, _NS, _NCH, _CB, _EC)
    y = _mm(x_pad, W0)
    p = _scat(y, idx_s, norm_c, selfc, b0)
    y = _mm2(p[0], p[1], W1)
    p = _scat(y, idx_s, norm_c, selfc, b1)
    y = _mm2(p[0], p[1], W2)
    p = _scat(y, idx_s, norm_c, selfc, b2)
    return _pool(p[0], p[1], batch_p)


# speedup vs baseline: 2.6702x; 2.6702x over previous
import functools

import jax
import jax.numpy as jnp
from jax import lax
from jax.experimental import pallas as pl
from jax.experimental.pallas import tpu as pltpu
from jax.experimental.pallas import tpu_sc as plsc

_f32 = jnp.float32
_i32 = jnp.int32

_NC, _NS = 2, 16
_D = 128
_B = 64
_NPAD = 10240
_RPT = _NPAD // _NS
_NQ = _NPAD // 16
_QPT = _NQ // _NS
_EC = 128
_K = 80
_CB = 8
_NCH = _K // _CB
_EPT = _K * _EC
_EPAD = _NC * _NS * _EPT
_NBLK = 8
_RB = _NPAD // _NBLK

_mesh = plsc.VectorSubcoreMesh(
    core_axis_name="c", subcore_axis_name="s", num_cores=_NC, num_subcores=_NS
)
_sc_params = pltpu.CompilerParams(needs_layout_passes=False)

def _bcast(v, lane):
    idx = jnp.full((16,), lane, _i32)
    return v.at[idx].get(mode="promise_in_bounds")


@functools.partial(
    pl.kernel,
    out_type=jax.ShapeDtypeStruct((_NC, _NQ, _D), _f32),
    mesh=_mesh,
    compiler_params=_sc_params,
    scratch_types=[
        pltpu.VMEM_SHARED((_NQ, _D), _f32),
        pltpu.VMEM((_K, _EC), _i32),
        pltpu.VMEM((_K, _EC), _i32),
        pltpu.VMEM((_K, _EC), _f32),
        pltpu.VMEM((_EC, _D), _f32),
        pltpu.VMEM((_QPT, _D), _f32),
    ],
)
def _deg(col_hbm, colq_hbm, ew_hbm, deg_out, acc16, colbuf, colqbuf, ewbuf,
         spread, degbuf):
    c = lax.axis_index("c")
    s = lax.axis_index("s")
    fiota = lax.iota(_i32, 16).astype(_f32)
    zeros16 = jnp.zeros((16,), _f32)

    def zdeg(i, carry):
        for g in range(8):
            degbuf[i, pl.ds(g * 16, 16)] = zeros16
        return carry

    lax.fori_loop(0, _QPT, zdeg, 0)
    pltpu.sync_copy(degbuf, acc16.at[pl.ds(s * _QPT, _QPT)])

    def zspread(i, carry):
        for g in range(8):
            spread[i, pl.ds(g * 16, 16)] = zeros16
        return carry

    lax.fori_loop(0, _EC, zspread, 0)
    plsc.subcore_barrier()

    pltpu.sync_copy(col_hbm.at[c, s], colbuf)
    pltpu.sync_copy(colq_hbm.at[c, s], colqbuf)
    pltpu.sync_copy(ew_hbm.at[c, s], ewbuf)

    def dstep(j, carry):
        for g in range(8):
            colg = colbuf[j, pl.ds(g * 16, 16)]
            ewg = ewbuf[j, pl.ds(g * 16, 16)]
            lowf = jnp.bitwise_and(colg, 15).astype(_f32)
            for lane in range(16):
                m = fiota == _bcast(lowf, lane)
                spread[g * 16 + lane, pl.ds(0, 16)] = jnp.where(
                    m, _bcast(ewg, lane), 0.0)
        pltpu.sync_copy(spread, acc16.at[colqbuf.at[j]], add=True)
        return carry

    lax.fori_loop(0, _K, dstep, 0)
    plsc.subcore_barrier()
    pltpu.sync_copy(acc16.at[pl.ds(s * _QPT, _QPT)], degbuf)
    pltpu.sync_copy(degbuf, deg_out.at[c, pl.ds(s * _QPT, _QPT)])


def _dinv_body(d_ref, dinv_ref, selfc_ref):
    deg = d_ref[0] + d_ref[1] + 2.0
    y = jnp.where(deg > 0, lax.rsqrt(jnp.where(deg > 0, deg, 1.0)), 0.0)
    dinv_ref[...] = y
    selfc_ref[...] = 2.0 * y * y


_dinv = pl.pallas_call(
    _dinv_body,
    out_shape=(
        jax.ShapeDtypeStruct((_NPAD // _D, _D), _f32),
        jax.ShapeDtypeStruct((_NPAD // _D, _D), _f32),
    ),
)


@functools.partial(
    pl.kernel,
    out_type=jax.ShapeDtypeStruct((_NC, _NS, _K, _EC), _f32),
    mesh=_mesh,
    compiler_params=_sc_params,
    scratch_types=[
        pltpu.VMEM((_K, _EC), _i32),
        pltpu.VMEM((_K, _EC), _i32),
        pltpu.VMEM((_K, _EC), _f32),
        pltpu.VMEM((_NPAD,), _f32),
        pltpu.VMEM((_K, _EC), _f32),
    ],
)
def _norm(row_hbm, col_hbm, ew_hbm, dinv_hbm, norm_out,
          rowbuf, colbuf, ewbuf, dinv_full, normbuf):
    c = lax.axis_index("c")
    s = lax.axis_index("s")
    pltpu.sync_copy(dinv_hbm, dinv_full)
    pltpu.sync_copy(row_hbm.at[c, s], rowbuf)
    pltpu.sync_copy(col_hbm.at[c, s], colbuf)
    pltpu.sync_copy(ew_hbm.at[c, s], ewbuf)

    def nstep(j, carry):
        for g in range(8):
            r = rowbuf[j, pl.ds(g * 16, 16)]
            cc = colbuf[j, pl.ds(g * 16, 16)]
            ew = ewbuf[j, pl.ds(g * 16, 16)]
            dr = plsc.load_gather(dinv_full, [r])
            dc = plsc.load_gather(dinv_full, [cc])
            normbuf[j, pl.ds(g * 16, 16)] = dr * ew * dc
        return carry

    lax.fori_loop(0, _K, nstep, 0)
    pltpu.sync_copy(normbuf, norm_out.at[c, s])


_DH = _D // 2


@functools.partial(
    pl.kernel,
    out_type=jax.ShapeDtypeStruct((_NC, _NPAD, _DH), _f32),
    mesh=_mesh,
    compiler_params=_sc_params,
    scratch_types=[
        pltpu.VMEM_SHARED((_NPAD, _DH), _f32),
        pltpu.VMEM_SHARED((_NPAD, _DH), _f32),
        pltpu.VMEM((2 * _CB, _EC), _i32),
        pltpu.VMEM((_CB, _EC), _f32),
        pltpu.VMEM((_EC, _DH), _f32),
        pltpu.VMEM((_EC, _DH), _f32),
        pltpu.VMEM((_RPT,), _f32),
        pltpu.VMEM((_D,), _f32),
        pltpu.SemaphoreType.DMA,
        pltpu.SemaphoreType.DMA,
        pltpu.SemaphoreType.DMA,
        pltpu.SemaphoreType.DMA,
    ],
)
def _scat(y_hbm, idx_hbm, norm_hbm, selfc_hbm, bias_hbm,
          p_hbm, acc, ysh, ibuf, nbuf, msg0, msg1, selfcbuf,
          biasbuf, gsem0, gsem1, ssem0, ssem1):
    c = lax.axis_index("c")
    s = lax.axis_index("s")
    pltpu.sync_copy(bias_hbm, biasbuf)
    bias_vs = [biasbuf[pl.ds(c * _DH + g * 16, 16)] for g in range(4)]
    pltpu.sync_copy(selfc_hbm.at[pl.ds(s * _RPT, _RPT)], selfcbuf)
    pltpu.sync_copy(y_hbm.at[c, pl.ds(s * _RPT, _RPT)],
                    ysh.at[pl.ds(s * _RPT, _RPT)])

    def ichunk(chunk, carry):
        base = s * _RPT + chunk * _EC
        pltpu.sync_copy(y_hbm.at[c, pl.ds(base, _EC)], msg0)

        def irow(gg, carry2):
            sv = selfcbuf[pl.ds(chunk * _EC + gg * 16, 16)]
            for lane in range(16):
                sc = _bcast(sv, lane)
                e = gg * 16 + lane
                for g in range(4):
                    msg0[e, pl.ds(g * 16, 16)] = (
                        msg0[e, pl.ds(g * 16, 16)] * sc + bias_vs[g])
            return carry2

        lax.fori_loop(0, _EC // 16, irow, 0)
        pltpu.sync_copy(msg0, acc.at[pl.ds(base, _EC)])
        return carry

    lax.fori_loop(0, _RPT // _EC, ichunk, 0)
    plsc.subcore_barrier()

    bufs = (msg0, msg1)
    gsems = (gsem0, gsem1)
    ssems = (ssem0, ssem1)

    def _scale(mb, t):
        def srow(gg, carry2):
            nv = nbuf[t, pl.ds(gg * 16, 16)]
            for lane in range(16):
                nb = _bcast(nv, lane)
                e = gg * 16 + lane
                for g in range(4):
                    mb[e, pl.ds(g * 16, 16)] = mb[e, pl.ds(g * 16, 16)] * nb
            return carry2

        lax.fori_loop(0, _EC // 16, srow, 0)

    def _slot(t, wait_scat, do_gather):
        b = t % 2
        b2 = (b + 1) % 2
        if wait_scat:
            pltpu.make_async_copy(
                bufs[b2], acc.at[ibuf.at[_CB]], ssems[b2]).wait()
        if do_gather:
            pltpu.async_copy(ysh.at[ibuf.at[t + 1]], bufs[b2], gsems[b2])
        pltpu.make_async_copy(ysh.at[ibuf.at[t]], bufs[b], gsems[b]).wait()
        _scale(bufs[b], t)
        pltpu.async_copy(bufs[b], acc.at[ibuf.at[_CB + t]], ssems[b],
                         add=True)

    def _chunk(q, first):
        if not first:
            pltpu.make_async_copy(msg1, acc.at[ibuf.at[_CB]], ssem1).wait()
        pltpu.sync_copy(idx_hbm.at[c, s, q], ibuf)
        pltpu.sync_copy(norm_hbm.at[c, s, q], nbuf)
        pltpu.async_copy(ysh.at[ibuf.at[0]], msg0, gsem0)

        def pair(t2, carry):
            t0 = 2 * t2

            def dyn_slot(t, b):
                b2 = (b + 1) % 2
                pltpu.make_async_copy(
                    bufs[b2], acc.at[ibuf.at[_CB]], ssems[b2]).wait()
                pltpu.async_copy(ysh.at[ibuf.at[t + 1]], bufs[b2],
                                 gsems[b2])
                pltpu.make_async_copy(
                    ysh.at[ibuf.at[t]], bufs[b], gsems[b]).wait()

                def srow(gg, carry2):
                    nv = nbuf[t, pl.ds(gg * 16, 16)]
                    mb = bufs[b]
                    for lane in range(16):
                        nb = _bcast(nv, lane)
                        e = gg * 16 + lane
                        for g in range(4):
                            mb[e, pl.ds(g * 16, 16)] = (
                                mb[e, pl.ds(g * 16, 16)] * nb)
                    return carry2

                lax.fori_loop(0, _EC // 16, srow, 0)
                pltpu.async_copy(bufs[b], acc.at[ibuf.at[_CB + t]], ssems[b],
                                 add=True)

            dyn_slot(t0, 0)
            dyn_slot(t0 + 1, 1)
            return carry

        _slot(0, False, True)
        _slot(1, True, True)
        lax.fori_loop(1, _CB // 2 - 1, pair, 0)
        _slot(_CB - 2, True, True)
        _slot(_CB - 1, True, False)

    _chunk(0, True)

    def qloop(q, carry):
        _chunk(q, False)
        return carry

    lax.fori_loop(1, _NCH, qloop, 0)
    pltpu.make_async_copy(msg1, acc.at[ibuf.at[_CB]], ssem1).wait()
    plsc.subcore_barrier()
    pltpu.sync_copy(acc.at[pl.ds(s * _RPT, _RPT)],
                    p_hbm.at[c, pl.ds(s * _RPT, _RPT)])


def _mm_body(x_ref, w_ref, o_ref):
    o_ref[...] = jnp.dot(x_ref[...], w_ref[...], preferred_element_type=_f32)


_mm = pl.pallas_call(
    _mm_body,
    grid=(_NBLK,),
    in_specs=[
        pl.BlockSpec((_RB, _D), lambda i: (i, 0)),
        pl.BlockSpec((_D, _D), lambda i: (0, 0)),
    ],
    out_specs=pl.BlockSpec((_RB, _D), lambda i: (i, 0)),
    out_shape=jax.ShapeDtypeStruct((_NPAD, _D), _f32),
)


def _mm2_body(p_ref, w_ref, o_ref):
    o_ref[...] = (
        jnp.dot(p_ref[0], w_ref[0, 0:_DH, :], preferred_element_type=_f32)
        + jnp.dot(p_ref[1], w_ref[0, _DH:_D, :], preferred_element_type=_f32))


_mm2 = pl.pallas_call(
    _mm2_body,
    grid=(_NBLK,),
    in_specs=[
        pl.BlockSpec((2, _RB, _DH), lambda i: (0, i, 0)),
        pl.BlockSpec((1, _D, _D), lambda i: (0, 0, 0)),
    ],
    out_specs=pl.BlockSpec((_RB, _D), lambda i: (i, 0)),
    out_shape=jax.ShapeDtypeStruct((_NPAD, _D), _f32),
)


def _pool_body(p_ref, b_ref, o_ref, cnt_ref):
    i = pl.program_id(0)

    @pl.when(i == 0)
    def _():
        o_ref[...] = jnp.zeros_like(o_ref)
        cnt_ref[...] = jnp.zeros_like(cnt_ref)

    ids = b_ref[0]
    oh = (lax.broadcasted_iota(_i32, (_B, _RB), 0) == ids).astype(_f32)
    o_ref[:, 0:_DH] += jnp.dot(oh, p_ref[0], preferred_element_type=_f32)
    o_ref[:, _DH:_D] += jnp.dot(oh, p_ref[1], preferred_element_type=_f32)
    cnt_ref[...] += jnp.dot(oh, jnp.ones((_RB, _DH), _f32),
                            preferred_element_type=_f32)

    @pl.when(i == pl.num_programs(0) - 1)
    def _():
        cnt = jnp.maximum(cnt_ref[...], 1.0)
        o_ref[:, 0:_DH] /= cnt
        o_ref[:, _DH:_D] /= cnt


_pool = pl.pallas_call(
    _pool_body,
    grid=(_NBLK,),
    in_specs=[
        pl.BlockSpec((2, _RB, _DH), lambda i: (0, i, 0)),
        pl.BlockSpec((1, 1, _RB), lambda i: (i, 0, 0)),
    ],
    out_specs=pl.BlockSpec((_B, _D), lambda i: (0, 0)),
    out_shape=jax.ShapeDtypeStruct((_B, _D), _f32),
    scratch_shapes=[pltpu.VMEM((_B, _DH), _f32)],
)


def kernel(x, edge_index, edge_attr, batch, W0, b0, W1, b1, W2, b2):
    n = x.shape[0]
    e = edge_index.shape[1]
    x_pad = jnp.pad(x, ((0, _NPAD - n), (0, 0)))
    ew = jnp.reshape(edge_attr, (-1,))
    pe = _EPAD - e
    row_s = jnp.pad(edge_index[0], (0, pe)).astype(_i32).reshape(
        _NC, _NS, _K, _EC)
    col_s = jnp.pad(edge_index[1], (0, pe)).astype(_i32).reshape(
        _NC, _NS, _K, _EC)
    ew_s = jnp.pad(ew, (0, pe)).reshape(_NC, _NS, _K, _EC)
    row4 = row_s.reshape(_NC, _NS, _NCH, _CB, _EC)
    col4 = col_s.reshape(_NC, _NS, _NCH, _CB, _EC)
    idx_s = jnp.concatenate([row4, col4], axis=3)
    batch_p = jnp.pad(batch.astype(_i32), (0, _NPAD - n),
                      constant_values=_B).reshape(_NBLK, 1, _RB)

    colq_s = lax.shift_right_logical(col_s, 4)
    deg2 = _deg(col_s, colq_s, ew_s)
    deg2 = deg2[:, :, :16].reshape(_NC, _NPAD // _D, _D)
    dinv, selfc = _dinv(deg2)
    dinv = dinv.reshape(_NPAD)
    selfc = selfc.reshape(_NPAD)
    norm_s = _norm(row_s, col_s, ew_s, dinv)
    norm_c = norm_s.reshape(_NC, _NS, _NCH, _CB, _EC)

    def halves(y):
        return jnp.stack([y[:, :_DH], y[:, _DH:]])

    w3 = lambda w: w.reshape(1, _D, _D)
    y2 = halves(_mm(x_pad, W0))
    p = _scat(y2, idx_s, norm_c, selfc, b0)
    y2 = halves(_mm2(p, w3(W1)))
    p = _scat(y2, idx_s, norm_c, selfc, b1)
    y2 = halves(_mm2(p, w3(W2)))
    p = _scat(y2, idx_s, norm_c, selfc, b2)
    return _pool(p, batch_p)


# speedup vs baseline: 3.0019x; 1.1242x over previous
import functools

import jax
import jax.numpy as jnp
from jax import lax
from jax.experimental import pallas as pl
from jax.experimental.pallas import tpu as pltpu
from jax.experimental.pallas import tpu_sc as plsc

_f32 = jnp.float32
_i32 = jnp.int32

_NC, _NS = 2, 16
_D = 128
_B = 64
_NPAD = 10240
_RPT = _NPAD // _NS
_NQ = _NPAD // 16
_QPT = _NQ // _NS
_EC = 128
_K = 80
_CB = 8
_NCH = _K // _CB
_EPT = _K * _EC
_EPAD = _NC * _NS * _EPT
_NBLK = 8
_RB = _NPAD // _NBLK

_mesh = plsc.VectorSubcoreMesh(
    core_axis_name="c", subcore_axis_name="s", num_cores=_NC, num_subcores=_NS
)
_sc_params = pltpu.CompilerParams(needs_layout_passes=False)

def _bcast(v, lane):
    idx = jnp.full((16,), lane, _i32)
    return v.at[idx].get(mode="promise_in_bounds")


@functools.partial(
    pl.kernel,
    out_type=jax.ShapeDtypeStruct((_NC, _NQ, _D), _f32),
    mesh=_mesh,
    compiler_params=_sc_params,
    scratch_types=[
        pltpu.VMEM_SHARED((_NQ, _D), _f32),
        pltpu.VMEM((_K, _EC), _i32),
        pltpu.VMEM((_K, _EC), _i32),
        pltpu.VMEM((_K, _EC), _f32),
        pltpu.VMEM((_EC, _D), _f32),
        pltpu.VMEM((_QPT, _D), _f32),
    ],
)
def _deg(col_hbm, colq_hbm, ew_hbm, deg_out, acc16, colbuf, colqbuf, ewbuf,
         spread, degbuf):
    c = lax.axis_index("c")
    s = lax.axis_index("s")
    fiota = lax.iota(_i32, 16).astype(_f32)
    zeros16 = jnp.zeros((16,), _f32)

    def zdeg(i, carry):
        for g in range(8):
            degbuf[i, pl.ds(g * 16, 16)] = zeros16
        return carry

    lax.fori_loop(0, _QPT, zdeg, 0)
    pltpu.sync_copy(degbuf, acc16.at[pl.ds(s * _QPT, _QPT)])

    def zspread(i, carry):
        for g in range(8):
            spread[i, pl.ds(g * 16, 16)] = zeros16
        return carry

    lax.fori_loop(0, _EC, zspread, 0)
    plsc.subcore_barrier()

    pltpu.sync_copy(col_hbm.at[c, s], colbuf)
    pltpu.sync_copy(colq_hbm.at[c, s], colqbuf)
    pltpu.sync_copy(ew_hbm.at[c, s], ewbuf)

    def dstep(j, carry):
        for g in range(8):
            colg = colbuf[j, pl.ds(g * 16, 16)]
            ewg = ewbuf[j, pl.ds(g * 16, 16)]
            lowf = jnp.bitwise_and(colg, 15).astype(_f32)
            for lane in range(16):
                m = fiota == _bcast(lowf, lane)
                spread[g * 16 + lane, pl.ds(0, 16)] = jnp.where(
                    m, _bcast(ewg, lane), 0.0)
        pltpu.sync_copy(spread, acc16.at[colqbuf.at[j]], add=True)
        return carry

    lax.fori_loop(0, _K, dstep, 0)
    plsc.subcore_barrier()
    pltpu.sync_copy(acc16.at[pl.ds(s * _QPT, _QPT)], degbuf)
    pltpu.sync_copy(degbuf, deg_out.at[c, pl.ds(s * _QPT, _QPT)])


def _dinv_body(d_ref, dinv_ref, selfc_ref):
    deg = d_ref[0] + d_ref[1] + 2.0
    y = jnp.where(deg > 0, lax.rsqrt(jnp.where(deg > 0, deg, 1.0)), 0.0)
    dinv_ref[...] = y
    selfc_ref[...] = 2.0 * y * y


_dinv = pl.pallas_call(
    _dinv_body,
    out_shape=(
        jax.ShapeDtypeStruct((_NPAD // _D, _D), _f32),
        jax.ShapeDtypeStruct((_NPAD // _D, _D), _f32),
    ),
)


@functools.partial(
    pl.kernel,
    out_type=jax.ShapeDtypeStruct((_NC, _NS, _K, _EC), _f32),
    mesh=_mesh,
    compiler_params=_sc_params,
    scratch_types=[
        pltpu.VMEM((_K, _EC), _i32),
        pltpu.VMEM((_K, _EC), _i32),
        pltpu.VMEM((_K, _EC), _f32),
        pltpu.VMEM((_NPAD,), _f32),
        pltpu.VMEM((_K, _EC), _f32),
    ],
)
def _norm(row_hbm, col_hbm, ew_hbm, dinv_hbm, norm_out,
          rowbuf, colbuf, ewbuf, dinv_full, normbuf):
    c = lax.axis_index("c")
    s = lax.axis_index("s")
    pltpu.sync_copy(dinv_hbm, dinv_full)
    pltpu.sync_copy(row_hbm.at[c, s], rowbuf)
    pltpu.sync_copy(col_hbm.at[c, s], colbuf)
    pltpu.sync_copy(ew_hbm.at[c, s], ewbuf)

    def nstep(j, carry):
        for g in range(8):
            r = rowbuf[j, pl.ds(g * 16, 16)]
            cc = colbuf[j, pl.ds(g * 16, 16)]
            ew = ewbuf[j, pl.ds(g * 16, 16)]
            dr = plsc.load_gather(dinv_full, [r])
            dc = plsc.load_gather(dinv_full, [cc])
            normbuf[j, pl.ds(g * 16, 16)] = dr * ew * dc
        return carry

    lax.fori_loop(0, _K, nstep, 0)
    pltpu.sync_copy(normbuf, norm_out.at[c, s])


_DH = _D // 2


@functools.partial(
    pl.kernel,
    out_type=jax.ShapeDtypeStruct((_NC, _NPAD, _DH), _f32),
    mesh=_mesh,
    compiler_params=_sc_params,
    scratch_types=[
        pltpu.VMEM_SHARED((_NPAD, _DH), _f32),
        pltpu.VMEM_SHARED((_NPAD, _DH), _f32),
        pltpu.VMEM((2 * _CB, _EC), _i32),
        pltpu.VMEM((_CB, _EC), _f32),
        pltpu.VMEM((_EC, _DH), _f32),
        pltpu.VMEM((_EC, _DH), _f32),
        pltpu.VMEM((_RPT,), _f32),
        pltpu.VMEM((_D,), _f32),
        pltpu.SemaphoreType.DMA,
        pltpu.SemaphoreType.DMA,
        pltpu.SemaphoreType.DMA,
        pltpu.SemaphoreType.DMA,
    ],
)
def _scat(y_hbm, idx_hbm, norm_hbm, selfc_hbm, bias_hbm,
          p_hbm, acc, ysh, ibuf, nbuf, msg0, msg1, selfcbuf,
          biasbuf, gsem0, gsem1, ssem0, ssem1):
    c = lax.axis_index("c")
    s = lax.axis_index("s")
    pltpu.sync_copy(bias_hbm, biasbuf)
    bias_vs = [biasbuf[pl.ds(c * _DH + g * 16, 16)] for g in range(4)]
    pltpu.sync_copy(selfc_hbm.at[pl.ds(s * _RPT, _RPT)], selfcbuf)
    pltpu.sync_copy(y_hbm.at[c, pl.ds(s * _RPT, _RPT)],
                    ysh.at[pl.ds(s * _RPT, _RPT)])

    def ichunk(chunk, carry):
        base = s * _RPT + chunk * _EC
        pltpu.sync_copy(y_hbm.at[c, pl.ds(base, _EC)], msg0)

        def irow(gg, carry2):
            sv = selfcbuf[pl.ds(chunk * _EC + gg * 16, 16)]
            for lane in range(16):
                sc = _bcast(sv, lane)
                e = gg * 16 + lane
                for g in range(4):
                    msg0[e, pl.ds(g * 16, 16)] = (
                        msg0[e, pl.ds(g * 16, 16)] * sc + bias_vs[g])
            return carry2

        lax.fori_loop(0, _EC // 16, irow, 0)
        pltpu.sync_copy(msg0, acc.at[pl.ds(base, _EC)])
        return carry

    lax.fori_loop(0, _RPT // _EC, ichunk, 0)

    zeros16 = jnp.zeros((16,), _f32)

    def zrow(e2, carry):
        for g in range(4):
            msg1[e2, pl.ds(g * 16, 16)] = zeros16
        return carry

    lax.fori_loop(0, _EC, zrow, 0)
    plsc.subcore_barrier()

    bufs = (msg0, msg1)
    gsems = (gsem0, gsem1)
    ssems = (ssem0, ssem1)

    pltpu.sync_copy(idx_hbm.at[c, s, 0], ibuf)
    pltpu.async_copy(msg1, acc.at[ibuf.at[_CB]], ssem1, add=True)

    def _slot(t, wait_scat, do_gather):
        b = t % 2
        b2 = (b + 1) % 2
        pltpu.make_async_copy(ysh.at[ibuf.at[t]], bufs[b], gsems[b]).wait()

        def srow(gg, carry2):
            nv = nbuf[t, pl.ds(gg * 16, 16)]
            mb = bufs[b]
            for lane in range(16):
                nb = _bcast(nv, lane)
                e = gg * 16 + lane
                for g in range(4):
                    mb[e, pl.ds(g * 16, 16)] = mb[e, pl.ds(g * 16, 16)] * nb
            return carry2

        lax.fori_loop(0, _EC // 16, srow, 0)
        if wait_scat:
            pltpu.make_async_copy(
                bufs[b2], acc.at[ibuf.at[_CB]], ssems[b2]).wait()
        if do_gather:
            pltpu.async_copy(ysh.at[ibuf.at[t + 1]], bufs[b2], gsems[b2])
        pltpu.async_copy(bufs[b], acc.at[ibuf.at[_CB + t]], ssems[b],
                         add=True)

    def qloop(q, carry):
        pltpu.make_async_copy(msg1, acc.at[ibuf.at[_CB]], ssem1).wait()
        pltpu.sync_copy(idx_hbm.at[c, s, q], ibuf)
        pltpu.sync_copy(norm_hbm.at[c, s, q], nbuf)
        pltpu.async_copy(ysh.at[ibuf.at[0]], msg0, gsem0)
        _slot(0, False, True)

        def pair(t2, carry2):
            t0 = 1 + 2 * t2

            def dyn_slot(t, b):
                b2 = (b + 1) % 2
                pltpu.make_async_copy(
                    ysh.at[ibuf.at[t]], bufs[b], gsems[b]).wait()

                def srow(gg, carry3):
                    nv = nbuf[t, pl.ds(gg * 16, 16)]
                    mb = bufs[b]
                    for lane in range(16):
                        nb = _bcast(nv, lane)
                        e = gg * 16 + lane
                        for g in range(4):
                            mb[e, pl.ds(g * 16, 16)] = (
                                mb[e, pl.ds(g * 16, 16)] * nb)
                    return carry3

                lax.fori_loop(0, _EC // 16, srow, 0)
                pltpu.make_async_copy(
                    bufs[b2], acc.at[ibuf.at[_CB]], ssems[b2]).wait()
                pltpu.async_copy(ysh.at[ibuf.at[t + 1]], bufs[b2],
                                 gsems[b2])
                pltpu.async_copy(bufs[b], acc.at[ibuf.at[_CB + t]],
                                 ssems[b], add=True)

            dyn_slot(t0, 1)
            dyn_slot(t0 + 1, 0)
            return carry2

        lax.fori_loop(0, _CB // 2 - 1, pair, 0)
        _slot(_CB - 1, True, False)
        return carry

    lax.fori_loop(0, _NCH, qloop, 0)
    pltpu.make_async_copy(msg1, acc.at[ibuf.at[_CB]], ssem1).wait()
    plsc.subcore_barrier()
    pltpu.sync_copy(acc.at[pl.ds(s * _RPT, _RPT)],
                    p_hbm.at[c, pl.ds(s * _RPT, _RPT)])


def _mm_body(x_ref, w_ref, o_ref):
    o_ref[...] = jnp.dot(x_ref[...], w_ref[...], preferred_element_type=_f32)


_mm = pl.pallas_call(
    _mm_body,
    grid=(_NBLK,),
    in_specs=[
        pl.BlockSpec((_RB, _D), lambda i: (i, 0)),
        pl.BlockSpec((_D, _D), lambda i: (0, 0)),
    ],
    out_specs=pl.BlockSpec((_RB, _D), lambda i: (i, 0)),
    out_shape=jax.ShapeDtypeStruct((_NPAD, _D), _f32),
)


def _mm2_body(p_ref, w_ref, o_ref):
    o_ref[...] = (
        jnp.dot(p_ref[0], w_ref[0, 0:_DH, :], preferred_element_type=_f32)
        + jnp.dot(p_ref[1], w_ref[0, _DH:_D, :], preferred_element_type=_f32))


_mm2 = pl.pallas_call(
    _mm2_body,
    grid=(_NBLK,),
    in_specs=[
        pl.BlockSpec((2, _RB, _DH), lambda i: (0, i, 0)),
        pl.BlockSpec((1, _D, _D), lambda i: (0, 0, 0)),
    ],
    out_specs=pl.BlockSpec((_RB, _D), lambda i: (i, 0)),
    out_shape=jax.ShapeDtypeStruct((_NPAD, _D), _f32),
)


def _pool_body(p_ref, b_ref, o_ref, cnt_ref):
    i = pl.program_id(0)

    @pl.when(i == 0)
    def _():
        o_ref[...] = jnp.zeros_like(o_ref)
        cnt_ref[...] = jnp.zeros_like(cnt_ref)

    ids = b_ref[0]
    oh = (lax.broadcasted_iota(_i32, (_B, _RB), 0) == ids).astype(_f32)
    o_ref[:, 0:_DH] += jnp.dot(oh, p_ref[0], preferred_element_type=_f32)
    o_ref[:, _DH:_D] += jnp.dot(oh, p_ref[1], preferred_element_type=_f32)
    cnt_ref[...] += jnp.dot(oh, jnp.ones((_RB, _DH), _f32),
                            preferred_element_type=_f32)

    @pl.when(i == pl.num_programs(0) - 1)
    def _():
        cnt = jnp.maximum(cnt_ref[...], 1.0)
        o_ref[:, 0:_DH] /= cnt
        o_ref[:, _DH:_D] /= cnt


_pool = pl.pallas_call(
    _pool_body,
    grid=(_NBLK,),
    in_specs=[
        pl.BlockSpec((2, _RB, _DH), lambda i: (0, i, 0)),
        pl.BlockSpec((1, 1, _RB), lambda i: (i, 0, 0)),
    ],
    out_specs=pl.BlockSpec((_B, _D), lambda i: (0, 0)),
    out_shape=jax.ShapeDtypeStruct((_B, _D), _f32),
    scratch_shapes=[pltpu.VMEM((_B, _DH), _f32)],
)


def kernel(x, edge_index, edge_attr, batch, W0, b0, W1, b1, W2, b2):
    n = x.shape[0]
    e = edge_index.shape[1]
    x_pad = jnp.pad(x, ((0, _NPAD - n), (0, 0)))
    ew = jnp.reshape(edge_attr, (-1,))
    pe = _EPAD - e
    row_s = jnp.pad(edge_index[0], (0, pe)).astype(_i32).reshape(
        _NC, _NS, _K, _EC)
    col_s = jnp.pad(edge_index[1], (0, pe)).astype(_i32).reshape(
        _NC, _NS, _K, _EC)
    ew_s = jnp.pad(ew, (0, pe)).reshape(_NC, _NS, _K, _EC)
    row4 = row_s.reshape(_NC, _NS, _NCH, _CB, _EC)
    col4 = col_s.reshape(_NC, _NS, _NCH, _CB, _EC)
    idx_s = jnp.concatenate([row4, col4], axis=3)
    batch_p = jnp.pad(batch.astype(_i32), (0, _NPAD - n),
                      constant_values=_B).reshape(_NBLK, 1, _RB)

    colq_s = lax.shift_right_logical(col_s, 4)
    deg2 = _deg(col_s, colq_s, ew_s)
    deg2 = deg2[:, :, :16].reshape(_NC, _NPAD // _D, _D)
    dinv, selfc = _dinv(deg2)
    dinv = dinv.reshape(_NPAD)
    selfc = selfc.reshape(_NPAD)
    norm_s = _norm(row_s, col_s, ew_s, dinv)
    norm_c = norm_s.reshape(_NC, _NS, _NCH, _CB, _EC)

    def halves(y):
        return jnp.stack([y[:, :_DH], y[:, _DH:]])

    w3 = lambda w: w.reshape(1, _D, _D)
    y2 = halves(_mm(x_pad, W0))
    p = _scat(y2, idx_s, norm_c, selfc, b0)
    y2 = halves(_mm2(p, w3(W1)))
    p = _scat(y2, idx_s, norm_c, selfc, b1)
    y2 = halves(_mm2(p, w3(W2)))
    p = _scat(y2, idx_s, norm_c, selfc, b2)
    return _pool(p, batch_p)


# speedup vs baseline: 3.0407x; 1.0129x over previous
import functools

import jax
import jax.numpy as jnp
from jax import lax
from jax.experimental import pallas as pl
from jax.experimental.pallas import tpu as pltpu
from jax.experimental.pallas import tpu_sc as plsc

_f32 = jnp.float32
_i32 = jnp.int32

_NC, _NS = 2, 16
_D = 128
_B = 64
_NPAD = 10240
_RPT = _NPAD // _NS
_NQ = _NPAD // 16
_QPT = _NQ // _NS
_EC = 128
_K = 80
_CB = 8
_NCH = _K // _CB
_EPT = _K * _EC
_EPAD = _NC * _NS * _EPT
_NBLK = 8
_RB = _NPAD // _NBLK

_mesh = plsc.VectorSubcoreMesh(
    core_axis_name="c", subcore_axis_name="s", num_cores=_NC, num_subcores=_NS
)
_sc_params = pltpu.CompilerParams(needs_layout_passes=False)

def _bcast(v, lane):
    idx = jnp.full((16,), lane, _i32)
    return v.at[idx].get(mode="promise_in_bounds")


_QW = 64
_NQ2 = _NPAD // _QW
_QPT2 = 16


@functools.partial(
    pl.kernel,
    out_type=jax.ShapeDtypeStruct((_NC, _NQ2, _QW), _f32),
    mesh=_mesh,
    compiler_params=_sc_params,
    scratch_types=[
        pltpu.VMEM_SHARED((_NQ2, _QW), _f32),
        pltpu.VMEM((_K, _EC), _i32),
        pltpu.VMEM((_K, _EC), _i32),
        pltpu.VMEM((_K, _EC), _f32),
        pltpu.VMEM((_EC, _QW), _f32),
        pltpu.VMEM((_QPT2, _QW), _f32),
    ],
)
def _deg(col_hbm, colq_hbm, ew_hbm, deg_out, acc16, colbuf, colqbuf, ewbuf,
         spread, degbuf):
    c = lax.axis_index("c")
    s = lax.axis_index("s")
    fiota = lax.iota(_i32, 16).astype(_f32)
    zeros16 = jnp.zeros((16,), _f32)

    def zdeg(i, carry):
        for g in range(_QW // 16):
            degbuf[i, pl.ds(g * 16, 16)] = zeros16
        return carry

    lax.fori_loop(0, _QPT2, zdeg, 0)

    @pl.when(s < _NQ2 // _QPT2)
    def _():
        pltpu.sync_copy(degbuf, acc16.at[pl.ds(s * _QPT2, _QPT2)])

    plsc.subcore_barrier()

    pltpu.sync_copy(col_hbm.at[c, s], colbuf)
    pltpu.sync_copy(colq_hbm.at[c, s], colqbuf)
    pltpu.sync_copy(ew_hbm.at[c, s], ewbuf)

    def dstep(j, carry):
        for g in range(8):
            colg = colbuf[j, pl.ds(g * 16, 16)]
            ewg = ewbuf[j, pl.ds(g * 16, 16)]
            lowf = jnp.bitwise_and(colg, _QW - 1).astype(_f32)
            for lane in range(16):
                low_bc = _bcast(lowf, lane)
                ew_bc = _bcast(ewg, lane)
                for g2 in range(_QW // 16):
                    m = (fiota + jnp.float32(16 * g2)) == low_bc
                    spread[g * 16 + lane, pl.ds(g2 * 16, 16)] = jnp.where(
                        m, ew_bc, 0.0)
        pltpu.sync_copy(spread, acc16.at[colqbuf.at[j]], add=True)
        return carry

    lax.fori_loop(0, _K, dstep, 0)
    plsc.subcore_barrier()

    @pl.when(s < _NQ2 // _QPT2)
    def _():
        pltpu.sync_copy(acc16.at[pl.ds(s * _QPT2, _QPT2)], degbuf)
        pltpu.sync_copy(degbuf, deg_out.at[c, pl.ds(s * _QPT2, _QPT2)])


def _dinv_body(d_ref, dinv_ref, selfc_ref):
    deg = d_ref[0] + d_ref[1] + 2.0
    y = jnp.where(deg > 0, lax.rsqrt(jnp.where(deg > 0, deg, 1.0)), 0.0)
    dinv_ref[...] = y
    selfc_ref[...] = 2.0 * y * y


_dinv = pl.pallas_call(
    _dinv_body,
    out_shape=(
        jax.ShapeDtypeStruct((_NPAD // _D, _D), _f32),
        jax.ShapeDtypeStruct((_NPAD // _D, _D), _f32),
    ),
)


@functools.partial(
    pl.kernel,
    out_type=jax.ShapeDtypeStruct((_NC, _NS, _K, _EC), _f32),
    mesh=_mesh,
    compiler_params=_sc_params,
    scratch_types=[
        pltpu.VMEM((_K, _EC), _i32),
        pltpu.VMEM((_K, _EC), _i32),
        pltpu.VMEM((_K, _EC), _f32),
        pltpu.VMEM((_NPAD,), _f32),
        pltpu.VMEM((_K, _EC), _f32),
    ],
)
def _norm(row_hbm, col_hbm, ew_hbm, dinv_hbm, norm_out,
          rowbuf, colbuf, ewbuf, dinv_full, normbuf):
    c = lax.axis_index("c")
    s = lax.axis_index("s")
    pltpu.sync_copy(dinv_hbm, dinv_full)
    pltpu.sync_copy(row_hbm.at[c, s], rowbuf)
    pltpu.sync_copy(col_hbm.at[c, s], colbuf)
    pltpu.sync_copy(ew_hbm.at[c, s], ewbuf)

    def nstep(j, carry):
        for g in range(8):
            r = rowbuf[j, pl.ds(g * 16, 16)]
            cc = colbuf[j, pl.ds(g * 16, 16)]
            ew = ewbuf[j, pl.ds(g * 16, 16)]
            dr = plsc.load_gather(dinv_full, [r])
            dc = plsc.load_gather(dinv_full, [cc])
            normbuf[j, pl.ds(g * 16, 16)] = dr * ew * dc
        return carry

    lax.fori_loop(0, _K, nstep, 0)
    pltpu.sync_copy(normbuf, norm_out.at[c, s])


_DH = _D // 2


@functools.partial(
    pl.kernel,
    out_type=jax.ShapeDtypeStruct((_NC, _NPAD, _DH), _f32),
    mesh=_mesh,
    compiler_params=_sc_params,
    scratch_types=[
        pltpu.VMEM_SHARED((_NPAD, _DH), _f32),
        pltpu.VMEM_SHARED((_NPAD, _DH), _f32),
        pltpu.VMEM((2 * _CB, _EC), _i32),
        pltpu.VMEM((_CB, _EC), _f32),
        pltpu.VMEM((_EC, _DH), _f32),
        pltpu.VMEM((_EC, _DH), _f32),
        pltpu.VMEM((_RPT,), _f32),
        pltpu.VMEM((_D,), _f32),
        pltpu.SemaphoreType.DMA,
        pltpu.SemaphoreType.DMA,
        pltpu.SemaphoreType.DMA,
        pltpu.SemaphoreType.DMA,
    ],
)
def _scat(y_hbm, idx_hbm, norm_hbm, selfc_hbm, bias_hbm,
          p_hbm, acc, ysh, ibuf, nbuf, msg0, msg1, selfcbuf,
          biasbuf, gsem0, gsem1, ssem0, ssem1):
    c = lax.axis_index("c")
    s = lax.axis_index("s")
    pltpu.sync_copy(bias_hbm, biasbuf)
    bias_vs = [biasbuf[pl.ds(c * _DH + g * 16, 16)] for g in range(4)]
    pltpu.sync_copy(selfc_hbm.at[pl.ds(s * _RPT, _RPT)], selfcbuf)
    pltpu.sync_copy(y_hbm.at[c, pl.ds(s * _RPT, _RPT)],
                    ysh.at[pl.ds(s * _RPT, _RPT)])

    def ichunk(chunk, carry):
        base = s * _RPT + chunk * _EC
        pltpu.sync_copy(y_hbm.at[c, pl.ds(base, _EC)], msg0)

        def irow(gg, carry2):
            sv = selfcbuf[pl.ds(chunk * _EC + gg * 16, 16)]
            for lane in range(16):
                sc = _bcast(sv, lane)
                e = gg * 16 + lane
                for g in range(4):
                    msg0[e, pl.ds(g * 16, 16)] = (
                        msg0[e, pl.ds(g * 16, 16)] * sc + bias_vs[g])
            return carry2

        lax.fori_loop(0, _EC // 16, irow, 0)
        pltpu.sync_copy(msg0, acc.at[pl.ds(base, _EC)])
        return carry

    lax.fori_loop(0, _RPT // _EC, ichunk, 0)

    zeros16 = jnp.zeros((16,), _f32)

    def zrow(e2, carry):
        for g in range(4):
            msg1[e2, pl.ds(g * 16, 16)] = zeros16
        return carry

    lax.fori_loop(0, _EC, zrow, 0)
    plsc.subcore_barrier()

    bufs = (msg0, msg1)
    gsems = (gsem0, gsem1)
    ssems = (ssem0, ssem1)

    pltpu.sync_copy(idx_hbm.at[c, s, 0], ibuf)
    pltpu.async_copy(msg1, acc.at[ibuf.at[_CB]], ssem1, add=True)

    def _slot(t, wait_scat, do_gather):
        b = t % 2
        b2 = (b + 1) % 2
        pltpu.make_async_copy(ysh.at[ibuf.at[t]], bufs[b], gsems[b]).wait()

        def srow(gg, carry2):
            nv = nbuf[t, pl.ds(gg * 16, 16)]
            mb = bufs[b]
            for lane in range(16):
                nb = _bcast(nv, lane)
                e = gg * 16 + lane
                for g in range(4):
                    mb[e, pl.ds(g * 16, 16)] = mb[e, pl.ds(g * 16, 16)] * nb
            return carry2

        lax.fori_loop(0, _EC // 16, srow, 0)
        if wait_scat:
            pltpu.make_async_copy(
                bufs[b2], acc.at[ibuf.at[_CB]], ssems[b2]).wait()
        if do_gather:
            pltpu.async_copy(ysh.at[ibuf.at[t + 1]], bufs[b2], gsems[b2])
        pltpu.async_copy(bufs[b], acc.at[ibuf.at[_CB + t]], ssems[b],
                         add=True)

    def qloop(q, carry):
        pltpu.make_async_copy(msg1, acc.at[ibuf.at[_CB]], ssem1).wait()
        pltpu.sync_copy(idx_hbm.at[c, s, q], ibuf)
        pltpu.sync_copy(norm_hbm.at[c, s, q], nbuf)
        pltpu.async_copy(ysh.at[ibuf.at[0]], msg0, gsem0)
        _slot(0, False, True)

        def pair(t2, carry2):
            t0 = 1 + 2 * t2

            def dyn_slot(t, b):
                b2 = (b + 1) % 2
                pltpu.make_async_copy(
                    ysh.at[ibuf.at[t]], bufs[b], gsems[b]).wait()

                def srow(gg, carry3):
                    nv = nbuf[t, pl.ds(gg * 16, 16)]
                    mb = bufs[b]
                    for lane in range(16):
                        nb = _bcast(nv, lane)
                        e = gg * 16 + lane
                        for g in range(4):
                            mb[e, pl.ds(g * 16, 16)] = (
                                mb[e, pl.ds(g * 16, 16)] * nb)
                    return carry3

                lax.fori_loop(0, _EC // 16, srow, 0)
                pltpu.make_async_copy(
                    bufs[b2], acc.at[ibuf.at[_CB]], ssems[b2]).wait()
                pltpu.async_copy(ysh.at[ibuf.at[t + 1]], bufs[b2],
                                 gsems[b2])
                pltpu.async_copy(bufs[b], acc.at[ibuf.at[_CB + t]],
                                 ssems[b], add=True)

            dyn_slot(t0, 1)
            dyn_slot(t0 + 1, 0)
            return carry2

        lax.fori_loop(0, _CB // 2 - 1, pair, 0)
        _slot(_CB - 1, True, False)
        return carry

    lax.fori_loop(0, _NCH, qloop, 0)
    pltpu.make_async_copy(msg1, acc.at[ibuf.at[_CB]], ssem1).wait()
    plsc.subcore_barrier()
    pltpu.sync_copy(acc.at[pl.ds(s * _RPT, _RPT)],
                    p_hbm.at[c, pl.ds(s * _RPT, _RPT)])


def _mm_body(x_ref, w_ref, o_ref):
    o_ref[...] = jnp.dot(x_ref[...], w_ref[...], preferred_element_type=_f32)


_mm = pl.pallas_call(
    _mm_body,
    grid=(_NBLK,),
    in_specs=[
        pl.BlockSpec((_RB, _D), lambda i: (i, 0)),
        pl.BlockSpec((_D, _D), lambda i: (0, 0)),
    ],
    out_specs=pl.BlockSpec((_RB, _D), lambda i: (i, 0)),
    out_shape=jax.ShapeDtypeStruct((_NPAD, _D), _f32),
)


def _mm2_body(p_ref, w_ref, o_ref):
    o_ref[...] = (
        jnp.dot(p_ref[0], w_ref[0, 0:_DH, :], preferred_element_type=_f32)
        + jnp.dot(p_ref[1], w_ref[0, _DH:_D, :], preferred_element_type=_f32))


_mm2 = pl.pallas_call(
    _mm2_body,
    grid=(_NBLK,),
    in_specs=[
        pl.BlockSpec((2, _RB, _DH), lambda i: (0, i, 0)),
        pl.BlockSpec((1, _D, _D), lambda i: (0, 0, 0)),
    ],
    out_specs=pl.BlockSpec((_RB, _D), lambda i: (i, 0)),
    out_shape=jax.ShapeDtypeStruct((_NPAD, _D), _f32),
)


def _pool_body(p_ref, b_ref, o_ref, cnt_ref):
    i = pl.program_id(0)

    @pl.when(i == 0)
    def _():
        o_ref[...] = jnp.zeros_like(o_ref)
        cnt_ref[...] = jnp.zeros_like(cnt_ref)

    ids = b_ref[0]
    oh = (lax.broadcasted_iota(_i32, (_B, _RB), 0) == ids).astype(_f32)
    o_ref[:, 0:_DH] += jnp.dot(oh, p_ref[0], preferred_element_type=_f32)
    o_ref[:, _DH:_D] += jnp.dot(oh, p_ref[1], preferred_element_type=_f32)
    cnt_ref[...] += jnp.dot(oh, jnp.ones((_RB, _DH), _f32),
                            preferred_element_type=_f32)

    @pl.when(i == pl.num_programs(0) - 1)
    def _():
        cnt = jnp.maximum(cnt_ref[...], 1.0)
        o_ref[:, 0:_DH] /= cnt
        o_ref[:, _DH:_D] /= cnt


_pool = pl.pallas_call(
    _pool_body,
    grid=(_NBLK,),
    in_specs=[
        pl.BlockSpec((2, _RB, _DH), lambda i: (0, i, 0)),
        pl.BlockSpec((1, 1, _RB), lambda i: (i, 0, 0)),
    ],
    out_specs=pl.BlockSpec((_B, _D), lambda i: (0, 0)),
    out_shape=jax.ShapeDtypeStruct((_B, _D), _f32),
    scratch_shapes=[pltpu.VMEM((_B, _DH), _f32)],
)


def kernel(x, edge_index, edge_attr, batch, W0, b0, W1, b1, W2, b2):
    n = x.shape[0]
    e = edge_index.shape[1]
    x_pad = jnp.pad(x, ((0, _NPAD - n), (0, 0)))
    ew = jnp.reshape(edge_attr, (-1,))
    pe = _EPAD - e
    row_s = jnp.pad(edge_index[0], (0, pe)).astype(_i32).reshape(
        _NC, _NS, _K, _EC)
    col_s = jnp.pad(edge_index[1], (0, pe)).astype(_i32).reshape(
        _NC, _NS, _K, _EC)
    ew_s = jnp.pad(ew, (0, pe)).reshape(_NC, _NS, _K, _EC)
    row4 = row_s.reshape(_NC, _NS, _NCH, _CB, _EC)
    col4 = col_s.reshape(_NC, _NS, _NCH, _CB, _EC)
    idx_s = jnp.concatenate([row4, col4], axis=3)
    batch_p = jnp.pad(batch.astype(_i32), (0, _NPAD - n),
                      constant_values=_B).reshape(_NBLK, 1, _RB)

    colq_s = lax.shift_right_logical(col_s, 6)
    deg2 = _deg(col_s, colq_s, ew_s)
    deg2 = deg2.reshape(_NC, _NPAD // _D, _D)
    dinv, selfc = _dinv(deg2)
    dinv = dinv.reshape(_NPAD)
    selfc = selfc.reshape(_NPAD)
    norm_s = _norm(row_s, col_s, ew_s, dinv)
    norm_c = norm_s.reshape(_NC, _NS, _NCH, _CB, _EC)

    def halves(y):
        return jnp.stack([y[:, :_DH], y[:, _DH:]])

    w3 = lambda w: w.reshape(1, _D, _D)
    y2 = halves(_mm(x_pad, W0))
    p = _scat(y2, idx_s, norm_c, selfc, b0)
    y2 = halves(_mm2(p, w3(W1)))
    p = _scat(y2, idx_s, norm_c, selfc, b1)
    y2 = halves(_mm2(p, w3(W2)))
    p = _scat(y2, idx_s, norm_c, selfc, b2)
    return _pool(p, batch_p)
